# baseline (speedup 1.0000x reference)
import functools

import jax
import jax.numpy as jnp
from jax import lax
from jax.experimental import pallas as pl
from jax.experimental.pallas import tpu as pltpu
from jax.experimental.pallas import tpu_sc as plsc

_N = 1048576
_C = 19
_NB = 1024
_HBINS = _C * 2 * _NB
_BL = 16384
_NW = 32
_PER_W = (_N * _C) // _NW
_CH = 4096


def _phase1_body(logits_ref, labels_ref, codes_ref):
    xt = logits_ref[...]
    lab = labels_ref[0]
    m = jnp.max(xt, axis=0, keepdims=True)
    ex = jnp.exp(xt - m)
    p = ex / jnp.sum(ex, axis=0, keepdims=True)
    cls = lax.broadcasted_iota(jnp.int32, (_C, _BL), 0)
    fg = lab == cls
    err = jnp.abs(fg.astype(jnp.float32) - p)
    b = jnp.clip((err * _NB).astype(jnp.int32), 0, _NB - 1)
    codes_ref[...] = b + jnp.where(fg, _NB, 0) + cls * (2 * _NB)


def _phase2_body(codes_hbm, out_hbm, hist_v, buf0, buf1, sem0, sem1):
    wid = lax.axis_index("s") * 2 + lax.axis_index("c")
    base = wid * _PER_W
    zeros16 = jnp.zeros((16,), jnp.int32)
    ones16 = jnp.ones((16,), jnp.int32)
    nch = _PER_W // _CH

    def zbody(i, carry):
        for k in range(16):
            hist_v[pl.ds(i * 256 + k * 16, 16)] = zeros16
        return carry

    lax.fori_loop(0, _HBINS // 256, zbody, 0)

    def ztail(i, carry):
        hist_v[pl.ds((_HBINS // 256) * 256 + i * 16, 16)] = zeros16
        return carry

    lax.fori_loop(0, (_HBINS % 256) // 16, ztail, 0)

    def scatter_all(buf):
        for j in range(_CH // 16):
            idx = buf[pl.ds(j * 16, 16)]
            plsc.addupdate_scatter(hist_v, [idx], ones16)

    pltpu.async_copy(codes_hbm.at[pl.ds(base, _CH)], buf0, sem0)

    def cbody(i, carry):
        c0 = 2 * i
        s1 = pl.multiple_of(base + (c0 + 1) * _CH, _CH)
        pltpu.async_copy(codes_hbm.at[pl.ds(s1, _CH)], buf1, sem1)
        s0 = pl.multiple_of(base + c0 * _CH, _CH)
        pltpu.make_async_copy(codes_hbm.at[pl.ds(s0, _CH)], buf0, sem0).wait()
        scatter_all(buf0)

        @pl.when(c0 + 2 < nch)
        def _():
            s2 = pl.multiple_of(base + (c0 + 2) * _CH, _CH)
            pltpu.async_copy(codes_hbm.at[pl.ds(s2, _CH)], buf0, sem0)

        pltpu.make_async_copy(codes_hbm.at[pl.ds(s1, _CH)], buf1, sem1).wait()
        scatter_all(buf1)
        return carry

    lax.fori_loop(0, nch // 2, cbody, 0)
    pltpu.sync_copy(hist_v, out_hbm.at[wid])


def _phase3_body(bg_ref, fg_ref, out_ref):
    bgs = bg_ref[0].astype(jnp.float32)
    fgs = fg_ref[0].astype(jnp.float32)
    for k in range(1, _NW):
        bgs = bgs + bg_ref[k].astype(jnp.float32)
        fgs = fgs + fg_ref[k].astype(jnp.float32)
    row = lax.broadcasted_iota(jnp.int32, (_NB, _NB), 0)
    col = lax.broadcasted_iota(jnp.int32, (_NB, _NB), 1)
    tri = (row >= col).astype(jnp.float32)
    tot = jnp.dot(bgs + fgs, tri, preferred_element_type=jnp.float32)
    pc = jnp.dot(fgs, tri, preferred_element_type=jnp.float32)
    g = pc[:, 0:1]
    jac = jnp.where(tot > 0,
                    1.0 - (g - pc) / jnp.maximum(g + tot - pc, 1.0),
                    0.0)
    lossc = (jnp.sum(jac, axis=1, keepdims=True) - 0.5 * jac[:, 0:1]) / _NB
    pres = (g > 0).astype(jnp.float32)
    num = jnp.sum(lossc * pres)
    den = jnp.maximum(jnp.sum(pres), 1.0)
    out_ref[...] = jnp.full((1, 1), num / den, jnp.float32)


def kernel(logits, labels):
    logits_t = jnp.swapaxes(logits, 0, 1)

    codes_t = pl.pallas_call(
        _phase1_body,
        grid=(_N // _BL,),
        in_specs=[
            pl.BlockSpec((_C, _BL), lambda i: (0, i)),
            pl.BlockSpec((1, 1, _BL), lambda i: (i, 0, 0)),
        ],
        out_specs=pl.BlockSpec((_C, _BL), lambda i: (0, i)),
        out_shape=jax.ShapeDtypeStruct((_C, _N), jnp.int32),
        compiler_params=pltpu.CompilerParams(
            dimension_semantics=("arbitrary",)),
    )(logits_t, labels.reshape(_N // _BL, 1, _BL))

    codes = jnp.swapaxes(codes_t, 0, 1)

    hist_kernel = functools.partial(
        pl.kernel,
        mesh=plsc.VectorSubcoreMesh(core_axis_name="c", subcore_axis_name="s"),
        compiler_params=pltpu.CompilerParams(needs_layout_passes=False),
        out_type=jax.ShapeDtypeStruct((_NW, _HBINS), jnp.int32),
        scratch_types=[
            pltpu.VMEM((_HBINS,), jnp.int32),
            pltpu.VMEM((_CH,), jnp.int32),
            pltpu.VMEM((_CH,), jnp.int32),
            pltpu.SemaphoreType.DMA,
            pltpu.SemaphoreType.DMA,
        ],
    )(_phase2_body)
    hists = hist_kernel(codes.reshape(_N * _C))

    h4 = hists.reshape(_NW, _C, 2, _NB)
    res = pl.pallas_call(
        _phase3_body,
        out_shape=jax.ShapeDtypeStruct((1, 1), jnp.float32),
    )(h4[:, :, 0, :], h4[:, :, 1, :])
    return res.reshape(())
